# baseline (speedup 1.0000x reference)
import functools

import jax
import jax.numpy as jnp
from jax import lax
from jax.experimental import pallas as pl
from jax.experimental.pallas import tpu as pltpu
from jax.experimental.pallas import tpu_sc as plsc

_RATE = 1.0
_MEAN = 0.5
_STD = 0.2
_UPPER = 10.0


def _mix_body(prior_ref, u_mix_ref, u_exp_ref, eps_ref, out_ref):
    p0 = prior_ref[0] / (prior_ref[0] + prior_ref[1])
    gauss = _MEAN + _STD * eps_ref[...]
    exp_s = -jnp.log1p(-u_exp_ref[...]) / _RATE
    out = jnp.where(u_mix_ref[...] >= p0, gauss, exp_s)
    out_ref[...] = jnp.clip(out, 0.0, _UPPER)


def _kernel_tc(u_mix, u_exp, eps_gauss, prior):
    R, C = u_mix.shape
    BC = 2048
    grid = (C // BC,)
    bspec = pl.BlockSpec((R, BC), lambda i: (0, i))
    return pl.pallas_call(
        _mix_body,
        grid=grid,
        in_specs=[
            pl.BlockSpec(memory_space=pltpu.SMEM),
            bspec,
            bspec,
            bspec,
        ],
        out_specs=bspec,
        out_shape=jax.ShapeDtypeStruct((R, C), jnp.float32),
    )(prior, u_mix, u_exp, eps_gauss)


_NC = 2
_NS = 16
_NW = _NC * _NS
_LANES = 16
_CHUNK = 8192

_G_COEFS = (
    0.0007252174248962018,
    0.417299795584964,
    -0.5729210622930205,
    0.15544599529659334,
)
_LN2 = 0.6931471805599453


def _neg_log(x):
    bits = plsc.bitcast(x, jnp.int32)
    t = bits.astype(jnp.float32) * jnp.float32(2.0 ** -23) - jnp.float32(127.0)
    m = plsc.bitcast((bits & 0x7FFFFF) | 0x3F800000, jnp.float32)
    r = m - 1.0
    g = jnp.float32(_G_COEFS[3])
    for c in (_G_COEFS[2], _G_COEFS[1], _G_COEFS[0]):
        g = g * r + jnp.float32(c)
    return jnp.float32(-_LN2) * (t + g)


def _sc_body(n_chunks, in_row_off, prior_hbm, umix_hbm, uexp_hbm, eps_hbm,
             out_hbm, pri_v, umix_v0, umix_v1, uexp_v0, uexp_v1, eps_v0,
             eps_v1, out_v0, out_v1, in_sem0, in_sem1, out_sem0, out_sem1):
    wid = lax.axis_index("s") * _NC + lax.axis_index("c")
    base_chunk = wid * n_chunks

    pltpu.sync_copy(prior_hbm, pri_v)
    pv = pri_v[...]
    rv = lax.rev(pv, (0,))
    lanes = lax.iota(jnp.int32, _LANES)
    num = jnp.where(lanes % 2 == 0, pv, rv)
    thresh = num / (pv + rv)

    in_sems = (in_sem0, in_sem1)
    out_sems = (out_sem0, out_sem1)
    umix_v = (umix_v0, umix_v1)
    uexp_v = (uexp_v0, uexp_v1)
    eps_v = (eps_v0, eps_v1)
    out_v = (out_v0, out_v1)

    def rowcol(ci):
        g = base_chunk + ci
        return g // _CHUNKS_PER_ROW, (g % _CHUNKS_PER_ROW) * _CHUNK

    def start_in(ci, slot):
        row, col = rowcol(ci)
        row = row + in_row_off
        sl = pl.ds(col, _CHUNK)
        return (
            pltpu.async_copy(umix_hbm.at[row, sl], umix_v[slot], in_sems[slot]),
            pltpu.async_copy(uexp_hbm.at[row, sl], uexp_v[slot], in_sems[slot]),
            pltpu.async_copy(eps_hbm.at[row, sl], eps_v[slot], in_sems[slot]),
        )

    def compute(slot):
        um_r = umix_v[slot]
        ue_r = uexp_v[slot]
        ep_r = eps_v[slot]
        ot_r = out_v[slot]

        @plsc.parallel_loop(0, _CHUNK // _LANES, unroll=8)
        def _(i):
            sl = pl.ds(i * _LANES, _LANES)
            exp_s = _neg_log(1.0 - ue_r[sl]) / _RATE
            gauss = _MEAN + _STD * ep_r[sl]
            res = jnp.where(um_r[sl] >= thresh, gauss, exp_s)
            ot_r[sl] = jnp.clip(res, 0.0, _UPPER)

    in_descs = [None, None]
    out_descs = [None, None]
    in_descs[0] = start_in(0, 0)
    for ci in range(n_chunks):
        slot = ci % 2
        if ci + 1 < n_chunks:
            in_descs[(ci + 1) % 2] = start_in(ci + 1, (ci + 1) % 2)
        for d in in_descs[slot]:
            d.wait()
        if out_descs[slot] is not None:
            out_descs[slot].wait()
        compute(slot)
        row, col = rowcol(ci)
        out_descs[slot] = pltpu.async_copy(
            out_v[slot], out_hbm.at[row, pl.ds(col, _CHUNK)], out_sems[slot]
        )
    for d in out_descs:
        if d is not None:
            d.wait()


_ROW_COLS = 32768
_CHUNKS_PER_ROW = _ROW_COLS // _CHUNK


def _sc_call(u_mix, u_exp, eps_gauss, prior, out_rows, in_row_off):
    C = u_mix.shape[1]
    n_chunks = out_rows * _CHUNKS_PER_ROW // _NW
    prior16 = jnp.tile(prior, _LANES // prior.shape[0])
    mesh = plsc.VectorSubcoreMesh(
        core_axis_name="c", subcore_axis_name="s",
        num_cores=_NC, num_subcores=_NS,
    )
    f = pl.kernel(
        functools.partial(_sc_body, n_chunks, in_row_off),
        out_type=jax.ShapeDtypeStruct((out_rows, C), jnp.float32),
        mesh=mesh,
        scratch_types=[pltpu.VMEM((_LANES,), jnp.float32)]
        + [pltpu.VMEM((_CHUNK,), jnp.float32)] * 8
        + [pltpu.SemaphoreType.DMA] * 4,
        compiler_params=pltpu.CompilerParams(needs_layout_passes=False),
    )
    return f(prior16, u_mix, u_exp, eps_gauss)


def _kernel_sc(u_mix, u_exp, eps_gauss, prior):
    return _sc_call(u_mix, u_exp, eps_gauss, prior, u_mix.shape[0], 0)


_R_TC = 80


def _kernel_hybrid(u_mix, u_exp, eps_gauss, prior):
    R, C = u_mix.shape
    r_sc = R - _R_TC
    sc_out = _sc_call(u_mix, u_exp, eps_gauss, prior, r_sc, _R_TC)
    BC = 4096
    bspec = pl.BlockSpec((_R_TC, BC), lambda i: (0, i))
    tc_out = pl.pallas_call(
        _mix_body,
        grid=(C // BC,),
        in_specs=[pl.BlockSpec(memory_space=pltpu.SMEM), bspec, bspec, bspec],
        out_specs=bspec,
        out_shape=jax.ShapeDtypeStruct((R, C), jnp.float32),
    )(prior, u_mix, u_exp, eps_gauss)
    return lax.dynamic_update_slice(tc_out, sc_out, (_R_TC, 0))


def kernel(u_mix, u_exp, eps_gauss, prior):
    return _kernel_tc(u_mix, u_exp, eps_gauss, prior)


# speedup vs baseline: 1.1142x; 1.1142x over previous
import functools

import jax
import jax.numpy as jnp
from jax import lax
from jax.experimental import pallas as pl
from jax.experimental.pallas import tpu as pltpu
from jax.experimental.pallas import tpu_sc as plsc

_RATE = 1.0
_MEAN = 0.5
_STD = 0.2
_UPPER = 10.0


def _mix_body(prior_ref, u_mix_ref, u_exp_ref, eps_ref, out_ref):
    p0 = prior_ref[0] / (prior_ref[0] + prior_ref[1])
    gauss = _MEAN + _STD * eps_ref[...]
    exp_s = -jnp.log1p(-u_exp_ref[...]) / _RATE
    out = jnp.where(u_mix_ref[...] >= p0, gauss, exp_s)
    out_ref[...] = jnp.clip(out, 0.0, _UPPER)


def _kernel_tc(u_mix, u_exp, eps_gauss, prior):
    R, C = u_mix.shape
    BC = 4096
    grid = (C // BC,)
    bspec = pl.BlockSpec((R, BC), lambda i: (0, i))
    return pl.pallas_call(
        _mix_body,
        grid=grid,
        in_specs=[
            pl.BlockSpec(memory_space=pltpu.SMEM),
            bspec,
            bspec,
            bspec,
        ],
        out_specs=bspec,
        out_shape=jax.ShapeDtypeStruct((R, C), jnp.float32),
    )(prior, u_mix, u_exp, eps_gauss)


_NC = 2
_NS = 16
_NW = _NC * _NS
_LANES = 16
_CHUNK = 8192

_G_COEFS = (
    0.0007252174248962018,
    0.417299795584964,
    -0.5729210622930205,
    0.15544599529659334,
)
_LN2 = 0.6931471805599453


def _neg_log(x):
    bits = plsc.bitcast(x, jnp.int32)
    t = bits.astype(jnp.float32) * jnp.float32(2.0 ** -23) - jnp.float32(127.0)
    m = plsc.bitcast((bits & 0x7FFFFF) | 0x3F800000, jnp.float32)
    r = m - 1.0
    g = jnp.float32(_G_COEFS[3])
    for c in (_G_COEFS[2], _G_COEFS[1], _G_COEFS[0]):
        g = g * r + jnp.float32(c)
    return jnp.float32(-_LN2) * (t + g)


def _sc_body(n_chunks, in_row_off, prior_hbm, umix_hbm, uexp_hbm, eps_hbm,
             out_hbm, pri_v, umix_v0, umix_v1, uexp_v0, uexp_v1, eps_v0,
             eps_v1, out_v0, out_v1, in_sem0, in_sem1, out_sem0, out_sem1):
    wid = lax.axis_index("s") * _NC + lax.axis_index("c")
    base_chunk = wid * n_chunks

    pltpu.sync_copy(prior_hbm, pri_v)
    pv = pri_v[...]
    rv = lax.rev(pv, (0,))
    lanes = lax.iota(jnp.int32, _LANES)
    num = jnp.where(lanes % 2 == 0, pv, rv)
    thresh = num / (pv + rv)

    in_sems = (in_sem0, in_sem1)
    out_sems = (out_sem0, out_sem1)
    umix_v = (umix_v0, umix_v1)
    uexp_v = (uexp_v0, uexp_v1)
    eps_v = (eps_v0, eps_v1)
    out_v = (out_v0, out_v1)

    def rowcol(ci):
        g = base_chunk + ci
        return g // _CHUNKS_PER_ROW, (g % _CHUNKS_PER_ROW) * _CHUNK

    def start_in(ci, slot):
        row, col = rowcol(ci)
        row = row + in_row_off
        sl = pl.ds(col, _CHUNK)
        return (
            pltpu.async_copy(umix_hbm.at[row, sl], umix_v[slot], in_sems[slot]),
            pltpu.async_copy(uexp_hbm.at[row, sl], uexp_v[slot], in_sems[slot]),
            pltpu.async_copy(eps_hbm.at[row, sl], eps_v[slot], in_sems[slot]),
        )

    def compute(slot):
        um_r = umix_v[slot]
        ue_r = uexp_v[slot]
        ep_r = eps_v[slot]
        ot_r = out_v[slot]

        @plsc.parallel_loop(0, _CHUNK // _LANES, unroll=8)
        def _(i):
            sl = pl.ds(i * _LANES, _LANES)
            exp_s = _neg_log(1.0 - ue_r[sl]) / _RATE
            gauss = _MEAN + _STD * ep_r[sl]
            res = jnp.where(um_r[sl] >= thresh, gauss, exp_s)
            ot_r[sl] = jnp.clip(res, 0.0, _UPPER)

    in_descs = [None, None]
    out_descs = [None, None]
    in_descs[0] = start_in(0, 0)
    for ci in range(n_chunks):
        slot = ci % 2
        if ci + 1 < n_chunks:
            in_descs[(ci + 1) % 2] = start_in(ci + 1, (ci + 1) % 2)
        for d in in_descs[slot]:
            d.wait()
        if out_descs[slot] is not None:
            out_descs[slot].wait()
        compute(slot)
        row, col = rowcol(ci)
        out_descs[slot] = pltpu.async_copy(
            out_v[slot], out_hbm.at[row, pl.ds(col, _CHUNK)], out_sems[slot]
        )
    for d in out_descs:
        if d is not None:
            d.wait()


_ROW_COLS = 32768
_CHUNKS_PER_ROW = _ROW_COLS // _CHUNK


def _sc_call(u_mix, u_exp, eps_gauss, prior, out_rows, in_row_off):
    C = u_mix.shape[1]
    n_chunks = out_rows * _CHUNKS_PER_ROW // _NW
    prior16 = jnp.tile(prior, _LANES // prior.shape[0])
    mesh = plsc.VectorSubcoreMesh(
        core_axis_name="c", subcore_axis_name="s",
        num_cores=_NC, num_subcores=_NS,
    )
    f = pl.kernel(
        functools.partial(_sc_body, n_chunks, in_row_off),
        out_type=jax.ShapeDtypeStruct((out_rows, C), jnp.float32),
        mesh=mesh,
        scratch_types=[pltpu.VMEM((_LANES,), jnp.float32)]
        + [pltpu.VMEM((_CHUNK,), jnp.float32)] * 8
        + [pltpu.SemaphoreType.DMA] * 4,
        compiler_params=pltpu.CompilerParams(needs_layout_passes=False),
    )
    return f(prior16, u_mix, u_exp, eps_gauss)


def _kernel_sc(u_mix, u_exp, eps_gauss, prior):
    return _sc_call(u_mix, u_exp, eps_gauss, prior, u_mix.shape[0], 0)


_R_TC = 80


def _kernel_hybrid(u_mix, u_exp, eps_gauss, prior):
    R, C = u_mix.shape
    r_sc = R - _R_TC
    sc_out = _sc_call(u_mix, u_exp, eps_gauss, prior, r_sc, _R_TC)
    BC = 4096
    bspec = pl.BlockSpec((_R_TC, BC), lambda i: (0, i))
    tc_out = pl.pallas_call(
        _mix_body,
        grid=(C // BC,),
        in_specs=[pl.BlockSpec(memory_space=pltpu.SMEM), bspec, bspec, bspec],
        out_specs=bspec,
        out_shape=jax.ShapeDtypeStruct((R, C), jnp.float32),
    )(prior, u_mix, u_exp, eps_gauss)
    return lax.dynamic_update_slice(tc_out, sc_out, (_R_TC, 0))


def kernel(u_mix, u_exp, eps_gauss, prior):
    return _kernel_tc(u_mix, u_exp, eps_gauss, prior)
